# baseline (speedup 1.0000x reference)
import jax
import jax.numpy as jnp
import numpy as np
from jax.experimental import pallas as pl
from jax.experimental.pallas import tpu as pltpu

_B, _T, _P, _D = 8, 32, 196, 768

_SKEY, _ = jax.random.split(jax.random.key(42), 2)
_S_SIZES = np.asarray(jax.random.choice(_SKEY, _T - 2, shape=(_B,))) + 1


def _tok_body(s_ref, tok_ref, mtok_ref, out_ref):
    b = pl.program_id(0)
    t = pl.program_id(1)
    masked = t >= s_ref[b]

    @pl.when(masked)
    def _():
        out_ref[...] = jnp.broadcast_to(mtok_ref[0, 0], out_ref.shape)

    @pl.when(jnp.logical_not(masked))
    def _():
        out_ref[...] = tok_ref[...]


def _pos_body(s_ref, out_ref):
    t_ids = jax.lax.broadcasted_iota(jnp.int32, (_B, _T), 1)
    out_ref[...] = (t_ids >= s_ref[...]).astype(jnp.int32)


def kernel(tokens, mask_token):
    s = jnp.asarray(_S_SIZES, dtype=jnp.int32)

    grid_spec = pltpu.PrefetchScalarGridSpec(
        num_scalar_prefetch=1,
        grid=(_B, _T),
        in_specs=[
            pl.BlockSpec(
                (1, 1, _P, _D),
                lambda b, t, s_ref: (b, jnp.minimum(t, s_ref[b] - 1), 0, 0),
            ),
            pl.BlockSpec((1, 1, _D), lambda b, t, s_ref: (b, 0, 0)),
        ],
        out_specs=pl.BlockSpec(
            (1, 1, _P, _D), lambda b, t, s_ref: (b, t, 0, 0)
        ),
    )
    masked_tokens = pl.pallas_call(
        _tok_body,
        grid_spec=grid_spec,
        out_shape=jax.ShapeDtypeStruct((_B, _T, _P, _D), tokens.dtype),
    )(s, tokens, mask_token)

    positions_i32 = pl.pallas_call(
        _pos_body,
        out_shape=jax.ShapeDtypeStruct((_B, _T), jnp.int32),
    )(s[:, None])
    return masked_tokens, positions_i32.astype(jnp.bool_)


# speedup vs baseline: 1.0029x; 1.0029x over previous
import jax
import jax.numpy as jnp
import numpy as np
from jax.experimental import pallas as pl
from jax.experimental.pallas import tpu as pltpu

_B, _T, _P, _D = 8, 32, 196, 768

_S_SIZES = np.array([5, 22, 30, 12, 11, 10, 1, 10], dtype=np.int32)


def _tok_body(s_ref, tok_ref, mtok_ref, out_ref):
    b = pl.program_id(0)
    t = pl.program_id(1)
    masked = t >= s_ref[b]

    @pl.when(masked)
    def _():
        out_ref[...] = jnp.broadcast_to(mtok_ref[0, 0], out_ref.shape)

    @pl.when(jnp.logical_not(masked))
    def _():
        out_ref[...] = tok_ref[...]


def _pos_body(s_ref, out_ref):
    t_ids = jax.lax.broadcasted_iota(jnp.int32, (_B, _T), 1)
    out_ref[...] = (t_ids >= s_ref[...]).astype(jnp.int32)


def kernel(tokens, mask_token):
    s = jnp.asarray(_S_SIZES, dtype=jnp.int32)

    grid_spec = pltpu.PrefetchScalarGridSpec(
        num_scalar_prefetch=1,
        grid=(_B, _T),
        in_specs=[
            pl.BlockSpec(
                (1, 1, _P, _D),
                lambda b, t, s_ref: (b, jnp.minimum(t, s_ref[b] - 1), 0, 0),
            ),
            pl.BlockSpec((1, 1, _D), lambda b, t, s_ref: (b, 0, 0)),
        ],
        out_specs=pl.BlockSpec(
            (1, 1, _P, _D), lambda b, t, s_ref: (b, t, 0, 0)
        ),
    )
    masked_tokens = pl.pallas_call(
        _tok_body,
        grid_spec=grid_spec,
        out_shape=jax.ShapeDtypeStruct((_B, _T, _P, _D), tokens.dtype),
    )(s, tokens, mask_token)

    positions_i32 = pl.pallas_call(
        _pos_body,
        out_shape=jax.ShapeDtypeStruct((_B, _T), jnp.int32),
    )(s[:, None])
    return masked_tokens, positions_i32.astype(jnp.bool_)


# speedup vs baseline: 1.2178x; 1.2143x over previous
import jax
import jax.numpy as jnp
import numpy as np
from jax.experimental import pallas as pl
from jax.experimental.pallas import tpu as pltpu

_B, _T, _P, _D = 8, 32, 196, 768

_S_SIZES = np.array([5, 22, 30, 12, 11, 10, 1, 10], dtype=np.int32)


_TBLK = 4


def _tok_body(s_ref, tok_ref, mtok_ref, out_ref):
    b = pl.program_id(0)
    tb = pl.program_id(1)
    s = s_ref[b]
    t0 = tb * _TBLK
    all_copy = t0 + _TBLK <= s
    all_mask = t0 >= s

    @pl.when(all_copy)
    def _():
        out_ref[...] = tok_ref[...]

    @pl.when(all_mask)
    def _():
        out_ref[...] = jnp.broadcast_to(mtok_ref[0, 0], out_ref.shape)

    @pl.when(jnp.logical_not(jnp.logical_or(all_copy, all_mask)))
    def _():
        t_ids = t0 + jax.lax.broadcasted_iota(jnp.int32, (1, _TBLK, 1, 1), 1)
        out_ref[...] = jnp.where(
            t_ids >= s,
            jnp.broadcast_to(mtok_ref[0, 0], out_ref.shape),
            tok_ref[...],
        )


def _pos_body(s_ref, out_ref):
    t_ids = jax.lax.broadcasted_iota(jnp.int32, (_B, _T), 1)
    out_ref[...] = (t_ids >= s_ref[...]).astype(jnp.int32)


def kernel(tokens, mask_token):
    s = jnp.asarray(_S_SIZES, dtype=jnp.int32)

    grid_spec = pltpu.PrefetchScalarGridSpec(
        num_scalar_prefetch=1,
        grid=(_B, _T // _TBLK),
        in_specs=[
            pl.BlockSpec(
                (1, _TBLK, _P, _D),
                lambda b, tb, s_ref: (
                    b,
                    jnp.minimum(tb, (s_ref[b] + _TBLK - 1) // _TBLK - 1),
                    0,
                    0,
                ),
            ),
            pl.BlockSpec((1, 1, _D), lambda b, tb, s_ref: (b, 0, 0)),
        ],
        out_specs=pl.BlockSpec(
            (1, _TBLK, _P, _D), lambda b, tb, s_ref: (b, tb, 0, 0)
        ),
    )
    masked_tokens = pl.pallas_call(
        _tok_body,
        grid_spec=grid_spec,
        out_shape=jax.ShapeDtypeStruct((_B, _T, _P, _D), tokens.dtype),
    )(s, tokens, mask_token)

    positions_i32 = pl.pallas_call(
        _pos_body,
        out_shape=jax.ShapeDtypeStruct((_B, _T), jnp.int32),
    )(s[:, None])
    return masked_tokens, positions_i32.astype(jnp.bool_)
